# baseline (speedup 1.0000x reference)
import functools

import jax
import jax.numpy as jnp
from jax import lax
from jax.experimental import pallas as pl
from jax.experimental.pallas import tpu as pltpu
from jax.experimental.pallas import tpu_sc as plsc

B, S, H, VOCAB = 64, 512, 768, 100000
PPW = 16
HS = H // 16
NB = 6
LEAD = 3
NI = B // NB
REM = B % NB


def _seg_bcast(sgf_p):
    sv = sgf_p[...].astype(jnp.float32)
    return [sv.at[jnp.full((16,), r, jnp.int32)].get(mode="promise_in_bounds")
            for r in range(PPW)]


def _add_posseg(rows_p, segb, posw, dloc):
    def hbody(h, c):
        off = pl.multiple_of(h * 16, 16)
        sl = pl.ds(off, 16)
        dh = dloc[sl]
        for r in range(PPW):
            plsc.addupdate(rows_p.at[r, sl], posw[r, sl] + segb[r] * dh)
        return c

    lax.fori_loop(0, HS, hbody, 0)


def _body(src, seg, wword, wpos, wseg, out,
          posw, dloc, wsg, idx, sgf, rows, *sems):
    gsem = sems[0:NB]
    ssem = sems[NB:2 * NB]
    isem = sems[2 * NB:3 * NB]
    info = plsc.get_sparse_core_info()
    nc = info.num_cores
    wid = lax.axis_index("s") * nc + lax.axis_index("c")
    pbase = wid * PPW
    psl = pl.ds(pbase, PPW)

    pltpu.sync_copy(wpos.at[psl], posw)
    pltpu.sync_copy(wseg, wsg)

    def prep_h(h, c):
        off = pl.multiple_of(h * 16, 16)
        sl = pl.ds(off, 16)
        s0h = wsg[0, sl]
        dloc[sl] = wsg[1, sl] - s0h
        for r in range(PPW):
            posw[r, sl] = posw[r, sl] + s0h
        return c

    lax.fori_loop(0, HS, prep_h, 0)

    def load_inputs(b, p):
        pltpu.async_copy(src.at[b, psl], idx.at[p], isem[p])
        pltpu.async_copy(seg.at[b, psl], sgf.at[p], isem[p])

    def wait_inputs(b, p):
        pltpu.make_async_copy(src.at[b, psl], idx.at[p], isem[p]).wait()
        pltpu.make_async_copy(seg.at[b, psl], sgf.at[p], isem[p]).wait()

    for t in range(NB):
        load_inputs(t, t)

    def start_gather(t, p):
        wait_inputs(t, p)
        pltpu.async_copy(wword.at[idx.at[p]], rows.at[p], gsem[p])

    def process(bq, q):
        segb = _seg_bcast(sgf.at[q])
        pltpu.make_async_copy(wword.at[idx.at[q]], rows.at[q], gsem[q]).wait()
        pl.when(bq + NB < B)(lambda: load_inputs(bq + NB, q))
        _add_posseg(rows.at[q], segb, posw, dloc)
        pltpu.async_copy(rows.at[q], out.at[bq, psl], ssem[q])

    def free_rows(p):
        pltpu.make_async_copy(rows.at[p], out.at[0, psl], ssem[p]).wait()

    def ibody(i, c):
        for p in range(NB):
            t = i * NB + p
            q = (p - LEAD) % NB

            pl.when(i >= 1)(lambda p=p: free_rows(p))
            start_gather(t, p)
            if p < LEAD:
                pl.when(i >= 1)(lambda t=t, q=q: process(t - LEAD, q))
            else:
                process(t - LEAD, q)
        return c

    lax.fori_loop(0, NI, ibody, 0)

    for p in range(REM):
        t = NI * NB + p
        free_rows(p)
        start_gather(t, p)
        process(t - LEAD, (p - LEAD) % NB)

    for k in range(LEAD):
        bq = B - LEAD + k
        process(bq, bq % NB)
    for p in range(NB):
        pltpu.make_async_copy(rows.at[p], out.at[0, psl], ssem[p]).wait()


_mesh = plsc.VectorSubcoreMesh(core_axis_name="c", subcore_axis_name="s")

_sc_call = functools.partial(
    pl.kernel,
    out_type=jax.ShapeDtypeStruct((B, S, H), jnp.float32),
    mesh=_mesh,
    scratch_types=[
        pltpu.VMEM((PPW, H), jnp.float32),
        pltpu.VMEM((H,), jnp.float32),
        pltpu.VMEM((2, H), jnp.float32),
        pltpu.VMEM((NB, PPW), jnp.int32),
        pltpu.VMEM((NB, PPW), jnp.int32),
        pltpu.VMEM((NB, PPW, H), jnp.float32),
    ] + [pltpu.SemaphoreType.DMA] * (3 * NB),
)(_body)


@jax.jit
def kernel(src, seg, W_word, W_pos, W_seg):
    return _sc_call(src, seg, W_word, W_pos, W_seg)


# speedup vs baseline: 1.3818x; 1.3818x over previous
import functools

import jax
import jax.numpy as jnp
from jax import lax
from jax.experimental import pallas as pl
from jax.experimental.pallas import tpu as pltpu
from jax.experimental.pallas import tpu_sc as plsc

B, S, H, VOCAB = 64, 512, 768, 100000
PPW = 16
HS = H // 16
NB = 4
LEAD = 2
NI = B // NB
REM = B % NB


def _seg_bcast(sgf_p):
    sv = sgf_p[...].astype(jnp.float32)
    return [sv.at[jnp.full((16,), r, jnp.int32)].get(mode="promise_in_bounds")
            for r in range(PPW)]


def _add_posseg(rows_p, segb, posw, dloc):
    def hbody(h, c):
        off = pl.multiple_of(h * 16, 16)
        sl = pl.ds(off, 16)
        dh = dloc[sl]
        for r in range(PPW):
            plsc.addupdate(rows_p.at[r, sl], posw[r, sl] + segb[r] * dh)
        return c

    lax.fori_loop(0, HS, hbody, 0)


def _body(src, seg, wword, wpos, wseg, out,
          posw, dloc, wsg, idx, sgf, rows, *sems):
    gsem = sems[0:NB]
    ssem = sems[NB:2 * NB]
    isem = sems[2 * NB:3 * NB]
    info = plsc.get_sparse_core_info()
    nc = info.num_cores
    wid = lax.axis_index("s") * nc + lax.axis_index("c")
    pbase = wid * PPW
    psl = pl.ds(pbase, PPW)

    pltpu.sync_copy(wpos.at[psl], posw)
    pltpu.sync_copy(wseg, wsg)

    def prep_h(h, c):
        off = pl.multiple_of(h * 16, 16)
        sl = pl.ds(off, 16)
        s0h = wsg[0, sl]
        dloc[sl] = wsg[1, sl] - s0h
        for r in range(PPW):
            posw[r, sl] = posw[r, sl] + s0h
        return c

    lax.fori_loop(0, HS, prep_h, 0)

    def load_inputs(b, p):
        pltpu.async_copy(src.at[b, psl], idx.at[p], isem[p])
        pltpu.async_copy(seg.at[b, psl], sgf.at[p], isem[p])

    def wait_inputs(b, p):
        pltpu.make_async_copy(src.at[b, psl], idx.at[p], isem[p]).wait()
        pltpu.make_async_copy(seg.at[b, psl], sgf.at[p], isem[p]).wait()

    for t in range(NB):
        load_inputs(t, t)

    def start_gather(t, p):
        wait_inputs(t, p)
        pltpu.async_copy(wword.at[idx.at[p]], rows.at[p], gsem[p])

    def process(bq, q):
        segb = _seg_bcast(sgf.at[q])
        pltpu.make_async_copy(wword.at[idx.at[q]], rows.at[q], gsem[q]).wait()
        pl.when(bq + NB < B)(lambda: load_inputs(bq + NB, q))
        _add_posseg(rows.at[q], segb, posw, dloc)
        pltpu.async_copy(rows.at[q], out.at[bq, psl], ssem[q])

    def free_rows(p):
        pltpu.make_async_copy(rows.at[p], out.at[0, psl], ssem[p]).wait()

    def ibody(i, c):
        for p in range(NB):
            t = i * NB + p
            q = (p - LEAD) % NB

            pl.when(i >= 1)(lambda p=p: free_rows(p))
            start_gather(t, p)
            if p < LEAD:
                pl.when(i >= 1)(lambda t=t, q=q: process(t - LEAD, q))
            else:
                process(t - LEAD, q)
        return c

    lax.fori_loop(0, NI, ibody, 0)

    for p in range(REM):
        t = NI * NB + p
        free_rows(p)
        start_gather(t, p)
        process(t - LEAD, (p - LEAD) % NB)

    for k in range(LEAD):
        bq = B - LEAD + k
        process(bq, bq % NB)
    for p in range(NB):
        pltpu.make_async_copy(rows.at[p], out.at[0, psl], ssem[p]).wait()


_mesh = plsc.VectorSubcoreMesh(core_axis_name="c", subcore_axis_name="s")

_sc_call = functools.partial(
    pl.kernel,
    out_type=jax.ShapeDtypeStruct((B, S, H), jnp.float32),
    mesh=_mesh,
    scratch_types=[
        pltpu.VMEM((PPW, H), jnp.float32),
        pltpu.VMEM((H,), jnp.float32),
        pltpu.VMEM((2, H), jnp.float32),
        pltpu.VMEM((NB, PPW), jnp.int32),
        pltpu.VMEM((NB, PPW), jnp.int32),
        pltpu.VMEM((NB, PPW, H), jnp.float32),
    ] + [pltpu.SemaphoreType.DMA] * (3 * NB),
)(_body)


@jax.jit
def kernel(src, seg, W_word, W_pos, W_seg):
    return _sc_call(src, seg, W_word, W_pos, W_seg)
